# baseline (speedup 1.0000x reference)
import jax
import jax.numpy as jnp
import numpy as np
from jax import lax
from jax.experimental import pallas as pl
from jax.experimental.pallas import tpu as pltpu
from jax.experimental.pallas import tpu_sc as plsc

B = 16384
HW = 49
CIN = 3
COUT = 20
ROW = HW * CIN
OUT_ROW = COUT * HW

NC, NS, L = 2, 16, 16
NW = NC * NS
B_PER_W = B // NW
GB = 4
VPG = GB * OUT_ROW // L
SUBS = 8
SG = GB * SUBS
NGROUPS = B_PER_W // SG

X_ROWS = B * ROW // 128
XW_WORDS = B_PER_W * ROW
XW_ROWS = XW_WORDS // 128
OUT_ROWS = B * OUT_ROW // 128
OW_ROWS = B_PER_W * OUT_ROW // 128
G_ROWS = SG * OUT_ROW // 128
G_VECS = SG * OUT_ROW // L


def _table():
    f = np.arange(GB * OUT_ROW)
    b = f // OUT_ROW
    r = f % OUT_ROW
    c = r // HW
    p = r % HW
    ch = np.where(c < 11, 0, np.where(c < 17, 1, 2))
    loc = np.where(c < 11, c, np.where(c < 17, c - 11, c - 17))
    idx = b * ROW + p * CIN + ch
    return (idx * 16 + loc).reshape(VPG, L).astype(np.int32)


_TAB = _table()


def _sc_body(x_hbm, tab_hbm, out_hbm, x_v, tab_v, out_v):
    wid = lax.axis_index("s") * NC + lax.axis_index("c")
    pltpu.sync_copy(tab_hbm, tab_v)
    pltpu.sync_copy(x_hbm.at[pl.ds(wid * XW_ROWS, XW_ROWS)], x_v)
    out_row0 = wid * OW_ROWS

    def group(g, carry):
        def sub(s, c1):
            off = (g * SG + s * GB) * ROW
            j0 = s * VPG

            def vec(v, c2):
                tv = tab_v[v]
                iv = lax.shift_right_logical(tv, 4) + off
                ir = lax.shift_right_logical(iv, 7)
                ic = lax.bitwise_and(iv, 127)
                xv = plsc.load_gather(x_v, [ir, ic])
                cv = lax.bitwise_and(tv, 15)
                j = j0 + v
                orow = lax.shift_right_logical(j, 3)
                ocol = lax.shift_left(lax.bitwise_and(j, 7), 4)
                out_v[orow, pl.ds(ocol, L)] = jnp.where(
                    xv == cv, jnp.float32(1.0), jnp.float32(0.0))
                return c2

            lax.fori_loop(0, VPG, vec, None, unroll=2)
            return c1

        lax.fori_loop(0, SUBS, sub, None)
        pltpu.sync_copy(out_v, out_hbm.at[pl.ds(out_row0 + g * G_ROWS, G_ROWS)])
        return carry

    lax.fori_loop(0, NGROUPS, group, None)


_SC_CALL = None


def _get_sc_call():
    global _SC_CALL
    if _SC_CALL is None:
        mesh = plsc.VectorSubcoreMesh(
            core_axis_name="c", subcore_axis_name="s",
            num_cores=NC, num_subcores=NS)
        _SC_CALL = pl.kernel(
            _sc_body,
            out_type=jax.ShapeDtypeStruct((OUT_ROWS, 128), jnp.float32),
            mesh=mesh,
            scratch_types=[
                pltpu.VMEM((XW_ROWS, 128), jnp.int32),
                pltpu.VMEM((VPG, L), jnp.int32),
                pltpu.VMEM((G_ROWS, 128), jnp.float32),
            ],
            compiler_params=pltpu.CompilerParams(
                needs_layout_passes=False, use_tc_tiling_on_sc=False),
        )
    return _SC_CALL


def kernel(x):
    xf = x.reshape(X_ROWS, 128)
    out = _get_sc_call()(xf, jnp.asarray(_TAB))
    return out.reshape(B, COUT, 7, 7)


# speedup vs baseline: 23.9529x; 23.9529x over previous
import jax
import jax.numpy as jnp
from jax import lax
from jax.experimental import pallas as pl
from jax.experimental.pallas import tpu as pltpu
from jax.experimental.pallas import tpu_sc as plsc

B = 16384
H = W = 7
CIN = 3
COUT = 20

NC, NS, L = 2, 16, 16
NW = NC * NS
BCH = 256
KCH = B // (NW * BCH)

_CH_LOC = [(0, c) for c in range(11)] + [(1, c) for c in range(6)] + \
          [(2, c) for c in range(3)]


def _sc_body(x_hbm, out_hbm, x_v0, x_v1, o_v0, o_v1,
             sx0, sx1, so0, so1):
    wid = lax.axis_index("s") * NC + lax.axis_index("c")
    xv = (x_v0, x_v1)
    sx = (sx0, sx1)
    slabs = (o_v0, o_v1)
    so = (so0, so1)

    dx = []
    for k in range(KCH):
        b0 = (wid * KCH + k) * BCH
        dx.append(pltpu.async_copy(
            x_hbm.at[:, :, :, pl.ds(b0, BCH)], xv[k], sx[k]))

    def compute_slab(x_k, slab, c):
        ge11 = (c >= 11).astype(jnp.int32)
        ge17 = (c >= 17).astype(jnp.int32)
        ch = ge11 + ge17
        loc = c - 11 * ge11 - 6 * ge17

        def hw(i, carry):
            h = i // W
            w = i - W * h

            def lane(j, c2):
                v = x_k[h, ch, w, pl.ds(j * L, L)]
                slab[h, w, pl.ds(j * L, L)] = jnp.where(
                    v == loc, jnp.float32(1.0), jnp.float32(0.0))
                return c2

            lax.fori_loop(0, BCH // L, lane, None, unroll=4)
            return carry

        lax.fori_loop(0, H * W, hw, None)

    for k in range(KCH):
        b0 = (wid * KCH + k) * BCH
        dx[k].wait()
        x_k = xv[k]

        def pair(i, carry, *, _x=x_k, _b0=b0, _k=k):
            for p in range(2):
                c = 2 * i + p
                dst = out_hbm.at[c, :, :, pl.ds(_b0, BCH)]

                def wait_prev(*, _p=p, _dst=dst):
                    pltpu.make_async_copy(slabs[_p], _dst, so[_p]).wait()

                if _k == 0:
                    pl.when(i > 0)(wait_prev)
                else:
                    wait_prev()
                compute_slab(_x, slabs[p], c)
                pltpu.async_copy(slabs[p], dst, so[p])
            return carry

        lax.fori_loop(0, COUT // 2, pair, None)

    b_last = (wid * KCH + KCH - 1) * BCH
    for p in range(2):
        pltpu.make_async_copy(
            slabs[p], out_hbm.at[18 + p, :, :, pl.ds(b_last, BCH)],
            so[p]).wait()


_SC_CALL = None


def _get_sc_call():
    global _SC_CALL
    if _SC_CALL is None:
        mesh = plsc.VectorSubcoreMesh(
            core_axis_name="c", subcore_axis_name="s",
            num_cores=NC, num_subcores=NS)
        _SC_CALL = pl.kernel(
            _sc_body,
            out_type=jax.ShapeDtypeStruct((COUT, H, W, B), jnp.float32),
            mesh=mesh,
            scratch_types=[
                pltpu.VMEM((H, CIN, W, BCH), jnp.int32),
                pltpu.VMEM((H, CIN, W, BCH), jnp.int32),
                pltpu.VMEM((H, W, BCH), jnp.float32),
                pltpu.VMEM((H, W, BCH), jnp.float32),
                pltpu.SemaphoreType.DMA,
                pltpu.SemaphoreType.DMA,
                pltpu.SemaphoreType.DMA,
                pltpu.SemaphoreType.DMA,
            ],
        )
    return _SC_CALL


def kernel(x):
    x_phys = jnp.transpose(x, (1, 3, 2, 0))
    out_phys = _get_sc_call()(x_phys)
    return jnp.transpose(out_phys, (3, 0, 1, 2))
